# initial kernel scaffold (unmeasured)
import jax
import jax.numpy as jnp
from jax import lax
from jax.experimental import pallas as pl
from jax.experimental.pallas import tpu as pltpu

N_DEV = 4


def _partial_matmul(dy, W):
    m, k = dy.shape
    n = W.shape[0]
    KB = 1024
    nk = k // KB

    def body(dy_ref, w_ref, out_ref, acc_ref):
        kk = pl.program_id(0)
        a = dy_ref[...].astype(jnp.bfloat16)
        b = w_ref[...].astype(jnp.bfloat16)
        part = lax.dot_general(
            a, b, (((1,), (1,)), ((), ())),
            preferred_element_type=jnp.float32,
        )

        @pl.when(kk == 0)
        def _():
            acc_ref[...] = part

        @pl.when(kk != 0)
        def _():
            acc_ref[...] += part

        @pl.when(kk == nk - 1)
        def _():
            out_ref[...] = acc_ref[...].astype(jnp.bfloat16)

    return pl.pallas_call(
        body,
        grid=(nk,),
        in_specs=[
            pl.BlockSpec((m, KB), lambda kk: (0, kk)),
            pl.BlockSpec((n, KB), lambda kk: (0, kk)),
        ],
        out_specs=pl.BlockSpec((m, n), lambda kk: (0, 0)),
        out_shape=jax.ShapeDtypeStruct((m, n), jnp.bfloat16),
        scratch_shapes=[pltpu.VMEM((m, n), jnp.float32)],
    )(dy, W)


def _ring_all_reduce(x):
    m, n = x.shape
    c = m // N_DEV

    def body(x_ref, out_ref, comm_ref, send_sems, recv_sems):
        p = lax.axis_index("i")
        left = lax.rem(p + N_DEV - 1, N_DEV)
        right = lax.rem(p + 1, N_DEV)

        barrier_sem = pltpu.get_barrier_semaphore()
        for nbr in (left, right):
            pl.semaphore_signal(
                barrier_sem, inc=1,
                device_id=(nbr,), device_id_type=pl.DeviceIdType.MESH,
            )
        pl.semaphore_wait(barrier_sem, 2)

        comm_ref[0, :, :] = x_ref[pl.ds(p * c, c), :]

        for h in range(2 * (N_DEV - 1)):
            s_slot = h % 2
            r_slot = (h + 1) % 2
            rdma = pltpu.make_async_remote_copy(
                src_ref=comm_ref.at[s_slot],
                dst_ref=comm_ref.at[r_slot],
                send_sem=send_sems.at[s_slot],
                recv_sem=recv_sems.at[r_slot],
                device_id=(right,),
                device_id_type=pl.DeviceIdType.MESH,
            )
            rdma.start()
            rdma.wait()

            if h < N_DEV - 1:
                idx = lax.rem(p + 2 * N_DEV - h - 1, N_DEV)
                comm_ref[r_slot, :, :] = (
                    comm_ref[r_slot, :, :] + x_ref[pl.ds(idx * c, c), :]
                )
                if h == N_DEV - 2:
                    out_ref[pl.ds(idx * c, c), :] = (
                        comm_ref[r_slot, :, :].astype(jnp.float32)
                    )
            else:
                idx = lax.rem(p + 2 * N_DEV - (h - (N_DEV - 1)), N_DEV)
                out_ref[pl.ds(idx * c, c), :] = (
                    comm_ref[r_slot, :, :].astype(jnp.float32)
                )

    return pl.pallas_call(
        body,
        out_shape=jax.ShapeDtypeStruct((m, n), jnp.float32),
        in_specs=[pl.BlockSpec(memory_space=pltpu.VMEM)],
        out_specs=pl.BlockSpec(memory_space=pltpu.VMEM),
        scratch_shapes=[
            pltpu.VMEM((2, c, n), jnp.bfloat16),
            pltpu.SemaphoreType.DMA((2,)),
            pltpu.SemaphoreType.DMA((2,)),
        ],
        compiler_params=pltpu.CompilerParams(collective_id=0),
    )(x)


def kernel(dy, W):
    partial = _partial_matmul(dy, W)
    return _ring_all_reduce(partial)


# baseline (device time: 268716 ns/iter reference)
import jax
import jax.numpy as jnp
from jax import lax
from jax.experimental import pallas as pl
from jax.experimental.pallas import tpu as pltpu

N_DEV = 4


def _partial_matmul(dy, W):
    m, k = dy.shape
    n = W.shape[0]
    MB = 1024
    KB = 1024
    nm = m // MB
    nk = k // KB

    def body(dy_ref, w_ref, out_ref, acc_ref):
        kk = pl.program_id(1)
        a = dy_ref[...].astype(jnp.bfloat16)
        b = w_ref[...].astype(jnp.bfloat16)
        part = lax.dot_general(
            a, b, (((1,), (1,)), ((), ())),
            preferred_element_type=jnp.float32,
        )

        @pl.when(kk == 0)
        def _():
            acc_ref[...] = part

        @pl.when(kk != 0)
        def _():
            acc_ref[...] += part

        @pl.when(kk == nk - 1)
        def _():
            out_ref[...] = acc_ref[...].astype(jnp.bfloat16)

    return pl.pallas_call(
        body,
        grid=(nm, nk),
        in_specs=[
            pl.BlockSpec((MB, KB), lambda mm, kk: (mm, kk)),
            pl.BlockSpec((n, KB), lambda mm, kk: (0, kk)),
        ],
        out_specs=pl.BlockSpec((MB, n), lambda mm, kk: (mm, 0)),
        out_shape=jax.ShapeDtypeStruct((m, n), jnp.bfloat16),
        scratch_shapes=[pltpu.VMEM((MB, n), jnp.float32)],
        compiler_params=pltpu.CompilerParams(
            vmem_limit_bytes=60 * 1024 * 1024,
        ),
    )(dy, W)


def _ring_all_reduce(x):
    m, n = x.shape
    c = m // N_DEV

    def body(x_ref, out_ref, comm_ref, send_sems, recv_sems):
        p = lax.axis_index("i")
        left = lax.rem(p + N_DEV - 1, N_DEV)
        right = lax.rem(p + 1, N_DEV)

        barrier_sem = pltpu.get_barrier_semaphore()
        for nbr in (left, right):
            pl.semaphore_signal(
                barrier_sem, inc=1,
                device_id=(nbr,), device_id_type=pl.DeviceIdType.MESH,
            )
        pl.semaphore_wait(barrier_sem, 2)

        comm_ref[0, :, :] = x_ref[pl.ds(p * c, c), :]

        for h in range(2 * (N_DEV - 1)):
            s_slot = h % 2
            r_slot = (h + 1) % 2
            rdma = pltpu.make_async_remote_copy(
                src_ref=comm_ref.at[s_slot],
                dst_ref=comm_ref.at[r_slot],
                send_sem=send_sems.at[s_slot],
                recv_sem=recv_sems.at[r_slot],
                device_id=(right,),
                device_id_type=pl.DeviceIdType.MESH,
            )
            rdma.start()
            rdma.wait()

            if h < N_DEV - 1:
                idx = lax.rem(p + 2 * N_DEV - h - 1, N_DEV)
                comm_ref[r_slot, :, :] = (
                    comm_ref[r_slot, :, :] + x_ref[pl.ds(idx * c, c), :]
                )
                if h == N_DEV - 2:
                    out_ref[pl.ds(idx * c, c), :] = (
                        comm_ref[r_slot, :, :].astype(jnp.float32)
                    )
            else:
                idx = lax.rem(p + 2 * N_DEV - (h - (N_DEV - 1)), N_DEV)
                out_ref[pl.ds(idx * c, c), :] = (
                    comm_ref[r_slot, :, :].astype(jnp.float32)
                )

    return pl.pallas_call(
        body,
        out_shape=jax.ShapeDtypeStruct((m, n), jnp.float32),
        in_specs=[pl.BlockSpec(memory_space=pltpu.VMEM)],
        out_specs=pl.BlockSpec(memory_space=pltpu.VMEM),
        scratch_shapes=[
            pltpu.VMEM((2, c, n), jnp.bfloat16),
            pltpu.SemaphoreType.DMA((2,)),
            pltpu.SemaphoreType.DMA((2,)),
        ],
        compiler_params=pltpu.CompilerParams(
            collective_id=0,
            vmem_limit_bytes=60 * 1024 * 1024,
        ),
    )(x)


def kernel(dy, W):
    partial = _partial_matmul(dy, W)
    return _ring_all_reduce(partial)


# device time: 225001 ns/iter; 1.1943x vs baseline; 1.1943x over previous
import jax
import jax.numpy as jnp
from jax import lax
from jax.experimental import pallas as pl
from jax.experimental.pallas import tpu as pltpu

N_DEV = 4


def _partial_matmul(dy, W):
    m, k = dy.shape
    n = W.shape[0]
    KB = 512
    NS = 512
    nk = k // KB

    def body(dy_ref, w_ref, out_ref, acc_ref):
        kk = pl.program_id(0)
        a = dy_ref[...].astype(jnp.bfloat16)
        for s in range(n // NS):
            b = w_ref[s * NS:(s + 1) * NS, :].astype(jnp.bfloat16)
            part = lax.dot_general(
                a, b, (((1,), (1,)), ((), ())),
                preferred_element_type=jnp.float32,
            )

            @pl.when(kk == 0)
            def _():
                acc_ref[:, s * NS:(s + 1) * NS] = part

            @pl.when(kk != 0)
            def _():
                acc_ref[:, s * NS:(s + 1) * NS] += part

        @pl.when(kk == nk - 1)
        def _():
            out_ref[...] = acc_ref[...].astype(jnp.bfloat16)

    return pl.pallas_call(
        body,
        grid=(nk,),
        in_specs=[
            pl.BlockSpec((m, KB), lambda kk: (0, kk)),
            pl.BlockSpec((n, KB), lambda kk: (0, kk)),
        ],
        out_specs=pl.BlockSpec((m, n), lambda kk: (0, 0)),
        out_shape=jax.ShapeDtypeStruct((m, n), jnp.bfloat16),
        scratch_shapes=[pltpu.VMEM((m, n), jnp.float32)],
        compiler_params=pltpu.CompilerParams(
            vmem_limit_bytes=60 * 1024 * 1024,
        ),
    )(dy, W)


def _ring_all_reduce(x):
    m, n = x.shape
    c = m // N_DEV
    ht = c // 2

    def body(x_ref, out_ref, comm_r, comm_l, send_r, recv_r, send_l, recv_l):
        p = lax.axis_index("i")
        left = lax.rem(p + N_DEV - 1, N_DEV)
        right = lax.rem(p + 1, N_DEV)

        barrier_sem = pltpu.get_barrier_semaphore()
        for nbr in (left, right):
            pl.semaphore_signal(
                barrier_sem, inc=1,
                device_id=(nbr,), device_id_type=pl.DeviceIdType.MESH,
            )
        pl.semaphore_wait(barrier_sem, 2)

        def top(idx):
            return pl.ds(idx * c, ht)

        def bot(idx):
            return pl.ds(idx * c + ht, ht)

        comm_r[0, :, :] = x_ref[top(p), :]
        comm_l[0, :, :] = x_ref[bot(p), :]

        for h in range(2 * (N_DEV - 1)):
            s_slot = h % 2
            r_slot = (h + 1) % 2
            rdma_r = pltpu.make_async_remote_copy(
                src_ref=comm_r.at[s_slot],
                dst_ref=comm_r.at[r_slot],
                send_sem=send_r.at[s_slot],
                recv_sem=recv_r.at[r_slot],
                device_id=(right,),
                device_id_type=pl.DeviceIdType.MESH,
            )
            rdma_l = pltpu.make_async_remote_copy(
                src_ref=comm_l.at[s_slot],
                dst_ref=comm_l.at[r_slot],
                send_sem=send_l.at[s_slot],
                recv_sem=recv_l.at[r_slot],
                device_id=(left,),
                device_id_type=pl.DeviceIdType.MESH,
            )
            rdma_r.start()
            rdma_l.start()
            rdma_r.wait()
            rdma_l.wait()

            if h < N_DEV - 1:
                idx_r = lax.rem(p + 2 * N_DEV - h - 1, N_DEV)
                comm_r[r_slot, :, :] = (
                    comm_r[r_slot, :, :] + x_ref[top(idx_r), :]
                )
                idx_l = lax.rem(p + h + 1, N_DEV)
                comm_l[r_slot, :, :] = (
                    comm_l[r_slot, :, :] + x_ref[bot(idx_l), :]
                )
                if h == N_DEV - 2:
                    out_ref[top(idx_r), :] = (
                        comm_r[r_slot, :, :].astype(jnp.float32)
                    )
                    out_ref[bot(idx_l), :] = (
                        comm_l[r_slot, :, :].astype(jnp.float32)
                    )
            else:
                g = h - (N_DEV - 1)
                idx_r = lax.rem(p + 2 * N_DEV - g, N_DEV)
                out_ref[top(idx_r), :] = (
                    comm_r[r_slot, :, :].astype(jnp.float32)
                )
                idx_l = lax.rem(p + g, N_DEV)
                out_ref[bot(idx_l), :] = (
                    comm_l[r_slot, :, :].astype(jnp.float32)
                )

    return pl.pallas_call(
        body,
        out_shape=jax.ShapeDtypeStruct((m, n), jnp.float32),
        in_specs=[pl.BlockSpec(memory_space=pltpu.VMEM)],
        out_specs=pl.BlockSpec(memory_space=pltpu.VMEM),
        scratch_shapes=[
            pltpu.VMEM((2, ht, n), jnp.bfloat16),
            pltpu.VMEM((2, ht, n), jnp.bfloat16),
            pltpu.SemaphoreType.DMA((2,)),
            pltpu.SemaphoreType.DMA((2,)),
            pltpu.SemaphoreType.DMA((2,)),
            pltpu.SemaphoreType.DMA((2,)),
        ],
        compiler_params=pltpu.CompilerParams(
            collective_id=0,
            vmem_limit_bytes=60 * 1024 * 1024,
        ),
    )(x)


def kernel(dy, W):
    partial = _partial_matmul(dy, W)
    return _ring_all_reduce(partial)


# device time: 189887 ns/iter; 1.4151x vs baseline; 1.1849x over previous
import jax
import jax.numpy as jnp
from jax import lax
from jax.experimental import pallas as pl
from jax.experimental.pallas import tpu as pltpu

N_DEV = 4
M = 2048
N = 2048
KSH = 8192
C = M // N_DEV
HT = C // 2
KB = 1024
NKB = KSH // KB
WB = 256
NWB = KSH // WB


def _fused_kernel(dy, W):
    def body(dy_hbm, w_hbm, out_hbm, w_bf, w_stage, dy_buf, acc, part_bf,
             comm_r, comm_l, outstage, w_sems, dy_sems, out_sems,
             send_r, recv_r, send_l, recv_l):
        p = lax.axis_index("i")
        left = lax.rem(p + N_DEV - 1, N_DEV)
        right = lax.rem(p + 1, N_DEV)
        ords = [p, left, right, lax.rem(p + 2, N_DEV)]

        barrier_sem = pltpu.get_barrier_semaphore()
        for nbr in (left, right):
            pl.semaphore_signal(
                barrier_sem, inc=1,
                device_id=(nbr,), device_id_type=pl.DeviceIdType.MESH,
            )
        pl.semaphore_wait(barrier_sem, 2)

        def dy_copy_dyn(row0, kb, slot):
            return pltpu.make_async_copy(
                dy_hbm.at[pl.ds(row0, C), pl.ds(kb * KB, KB)],
                dy_buf.at[slot],
                dy_sems.at[slot],
            )

        def w_copy(i):
            return pltpu.make_async_copy(
                w_hbm.at[:, pl.ds(i * WB, WB)],
                w_stage.at[i % 2],
                w_sems.at[i % 2],
            )

        def hop(h):
            s, r = h % 2, (h + 1) % 2
            rr = pltpu.make_async_remote_copy(
                src_ref=comm_r.at[s], dst_ref=comm_r.at[r],
                send_sem=send_r.at[s], recv_sem=recv_r.at[r],
                device_id=(right,), device_id_type=pl.DeviceIdType.MESH,
            )
            ll = pltpu.make_async_remote_copy(
                src_ref=comm_l.at[s], dst_ref=comm_l.at[r],
                send_sem=send_l.at[s], recv_sem=recv_l.at[r],
                device_id=(left,), device_id_type=pl.DeviceIdType.MESH,
            )
            return rr, ll

        pending_out = [None, None]

        def store_half(src_val_bf16, row0, slot):
            if pending_out[slot] is not None:
                pending_out[slot].wait()
            outstage[slot, :, :] = src_val_bf16.astype(jnp.float32)
            d = pltpu.make_async_copy(
                outstage.at[slot],
                out_hbm.at[pl.ds(row0, HT), :],
                out_sems.at[slot],
            )
            d.start()
            pending_out[slot] = d

        dy_copy_dyn(ords[0] * C, 0, 0).start()
        dy_copy_dyn(ords[0] * C, 1, 1).start()
        w_copy(0).start()
        w_copy(1).start()

        def w_copy_dyn(i):
            return pltpu.make_async_copy(
                w_hbm.at[:, pl.ds(i * WB, WB)],
                w_stage.at[lax.rem(i, 2)],
                w_sems.at[lax.rem(i, 2)],
            )

        for c in range(N_DEV):
            row_this = ords[c] * C
            row_next = ords[c + 1] * C if c + 1 < N_DEV else ords[c] * C

            def kb_body(kb, _, c=c, row_this=row_this, row_next=row_next):
                g = c * NKB + kb
                slot = lax.rem(g, 2)
                if c == 0:
                    for j in range(KB // WB):
                        i = (KB // WB) * kb + j
                        w_copy_dyn(i).wait()
                        w_bf[:, pl.ds(i * WB, WB)] = (
                            w_stage[lax.rem(i, 2), :, :].astype(jnp.bfloat16)
                        )

                        @pl.when(i + 2 < NWB)
                        def _():
                            w_copy_dyn(i + 2).start()

                dy_copy_dyn(row_this, kb, slot).wait()
                a = dy_buf[slot, :, :].astype(jnp.bfloat16)

                @pl.when(kb < NKB - 2)
                def _():
                    dy_copy_dyn(row_this, kb + 2, slot).start()

                if c + 1 < N_DEV:
                    @pl.when(kb >= NKB - 2)
                    def _():
                        dy_copy_dyn(row_next, kb - (NKB - 2), slot).start()

                b = w_bf[:, pl.ds(kb * KB, KB)]
                part = lax.dot_general(
                    a, b, (((1,), (1,)), ((), ())),
                    preferred_element_type=jnp.float32,
                )

                @pl.when(kb == 0)
                def _():
                    acc[...] = part

                @pl.when(kb != 0)
                def _():
                    acc[...] += part

                return 0

            lax.fori_loop(0, NKB, kb_body, 0)

            if c == 0:
                comm_r[0, :, :] = acc[0:HT, :].astype(jnp.bfloat16)
                comm_l[0, :, :] = acc[HT:C, :].astype(jnp.bfloat16)
                r0, l0 = hop(0)
                r0.start()
                l0.start()
            else:
                part_bf[(c - 1) * C:c * C, :] = acc[...].astype(jnp.bfloat16)

            if c == 2:
                r0, l0 = hop(0)
                r0.wait()
                l0.wait()
                comm_r[1, :, :] += part_bf[0:HT, :]
                comm_l[1, :, :] += part_bf[C + HT:2 * C, :]
                r1, l1 = hop(1)
                r1.start()
                l1.start()

        r1, l1 = hop(1)
        r1.wait()
        l1.wait()
        comm_r[0, :, :] += part_bf[2 * C:2 * C + HT, :]
        comm_l[0, :, :] += part_bf[2 * C + HT:3 * C, :]
        r2, l2 = hop(2)
        r2.start()
        l2.start()
        r2.wait()
        l2.wait()
        comm_r[1, :, :] += part_bf[C:C + HT, :]
        comm_l[1, :, :] += part_bf[HT:C, :]
        store_half(comm_r[1, :, :], lax.rem(p + 1, N_DEV) * C, 0)
        store_half(comm_l[1, :, :], lax.rem(p + N_DEV - 1, N_DEV) * C + HT, 1)

        for h in range(N_DEV - 1, 2 * (N_DEV - 1)):
            g = h - (N_DEV - 1)
            rr, ll = hop(h)
            rr.start()
            ll.start()
            rr.wait()
            ll.wait()
            rs = (h + 1) % 2
            idx_r = lax.rem(p + 2 * N_DEV - g, N_DEV)
            idx_l = lax.rem(p + g, N_DEV)
            store_half(comm_r[rs, :, :], idx_r * C, 0)
            store_half(comm_l[rs, :, :], idx_l * C + HT, 1)

        pending_out[0].wait()
        pending_out[1].wait()

    return pl.pallas_call(
        body,
        out_shape=jax.ShapeDtypeStruct((M, N), jnp.float32),
        in_specs=[
            pl.BlockSpec(memory_space=pl.ANY),
            pl.BlockSpec(memory_space=pl.ANY),
        ],
        out_specs=pl.BlockSpec(memory_space=pl.ANY),
        scratch_shapes=[
            pltpu.VMEM((N, KSH), jnp.bfloat16),
            pltpu.VMEM((2, N, WB), jnp.float32),
            pltpu.VMEM((2, C, KB), jnp.float32),
            pltpu.VMEM((C, N), jnp.float32),
            pltpu.VMEM((3 * C, N), jnp.bfloat16),
            pltpu.VMEM((2, HT, N), jnp.bfloat16),
            pltpu.VMEM((2, HT, N), jnp.bfloat16),
            pltpu.VMEM((2, HT, N), jnp.float32),
            pltpu.SemaphoreType.DMA((2,)),
            pltpu.SemaphoreType.DMA((2,)),
            pltpu.SemaphoreType.DMA((2,)),
            pltpu.SemaphoreType.DMA((2,)),
            pltpu.SemaphoreType.DMA((2,)),
            pltpu.SemaphoreType.DMA((2,)),
            pltpu.SemaphoreType.DMA((2,)),
        ],
        compiler_params=pltpu.CompilerParams(
            collective_id=0,
            vmem_limit_bytes=63 * 1024 * 1024,
        ),
    )(dy, W)


def kernel(dy, W):
    return _fused_kernel(dy, W)


# device time: 182674 ns/iter; 1.4710x vs baseline; 1.0395x over previous
import os

import jax
import jax.numpy as jnp
from jax import lax
from jax.experimental import pallas as pl
from jax.experimental.pallas import tpu as pltpu

_SKIP_RING = bool(os.environ.get("SKIP_RING"))

N_DEV = 4
M = 2048
N = 2048
KSH = 8192
C = M // N_DEV
HT = C // 2
KB = 1024
NKB = KSH // KB
WB = 256
NWB = KSH // WB


def _fused_kernel(dy, W):
    def body(dy_hbm, w_hbm, out_hbm, w_bf, w_stage, dy_buf, acc, part_bf,
             comm_r, comm_l, outstage, w_sems, dy_sems, out_sems,
             send_r, recv_r, send_l, recv_l):
        p = lax.axis_index("i")
        left = lax.rem(p + N_DEV - 1, N_DEV)
        right = lax.rem(p + 1, N_DEV)
        ords = [p, left, right, lax.rem(p + 2, N_DEV)]

        barrier_sem = pltpu.get_barrier_semaphore()
        for nbr in (left, right):
            pl.semaphore_signal(
                barrier_sem, inc=1,
                device_id=(nbr,), device_id_type=pl.DeviceIdType.MESH,
            )
        pl.semaphore_wait(barrier_sem, 2)

        def dy_copy_dyn(row0, kb, slot):
            return pltpu.make_async_copy(
                dy_hbm.at[pl.ds(row0, C), pl.ds(kb * KB, KB)],
                dy_buf.at[slot],
                dy_sems.at[slot],
            )

        def w_copy(i):
            return pltpu.make_async_copy(
                w_hbm.at[:, pl.ds(i * WB, WB)],
                w_stage.at[i % 2],
                w_sems.at[i % 2],
            )

        SB = HT // 2

        def hop(h):
            s, r = h % 2, (h + 1) % 2
            descs = []
            for u in (0, 1):
                descs.append(pltpu.make_async_remote_copy(
                    src_ref=comm_r.at[s, pl.ds(u * SB, SB)],
                    dst_ref=comm_r.at[r, pl.ds(u * SB, SB)],
                    send_sem=send_r.at[s, u], recv_sem=recv_r.at[r, u],
                    device_id=(right,), device_id_type=pl.DeviceIdType.MESH,
                ))
                descs.append(pltpu.make_async_remote_copy(
                    src_ref=comm_l.at[s, pl.ds(u * SB, SB)],
                    dst_ref=comm_l.at[r, pl.ds(u * SB, SB)],
                    send_sem=send_l.at[s, u], recv_sem=recv_l.at[r, u],
                    device_id=(left,), device_id_type=pl.DeviceIdType.MESH,
                ))
            return descs

        pending_out = [None, None]

        def store_half(src_val_bf16, row0, slot):
            if pending_out[slot] is not None:
                pending_out[slot].wait()
            outstage[slot, :, :] = src_val_bf16.astype(jnp.float32)
            d = pltpu.make_async_copy(
                outstage.at[slot],
                out_hbm.at[pl.ds(row0, HT), :],
                out_sems.at[slot],
            )
            d.start()
            pending_out[slot] = d

        dy_copy_dyn(ords[0] * C, 0, 0).start()
        dy_copy_dyn(ords[0] * C, 1, 1).start()
        w_copy(0).start()
        w_copy(1).start()

        def w_copy_dyn(i):
            return pltpu.make_async_copy(
                w_hbm.at[:, pl.ds(i * WB, WB)],
                w_stage.at[lax.rem(i, 2)],
                w_sems.at[lax.rem(i, 2)],
            )

        for c in range(N_DEV):
            row_this = ords[c] * C
            row_next = ords[c + 1] * C if c + 1 < N_DEV else ords[c] * C

            def kb_body(kb, _, c=c, row_this=row_this, row_next=row_next):
                g = c * NKB + kb
                slot = lax.rem(g, 2)
                if c == 0:
                    for j in range(KB // WB):
                        i = (KB // WB) * kb + j
                        w_copy_dyn(i).wait()
                        w_bf[:, pl.ds(i * WB, WB)] = (
                            w_stage[lax.rem(i, 2), :, :].astype(jnp.bfloat16)
                        )

                        @pl.when(i + 2 < NWB)
                        def _():
                            w_copy_dyn(i + 2).start()

                dy_copy_dyn(row_this, kb, slot).wait()
                a = dy_buf[slot, :, :].astype(jnp.bfloat16)

                @pl.when(kb < NKB - 2)
                def _():
                    dy_copy_dyn(row_this, kb + 2, slot).start()

                if c + 1 < N_DEV:
                    @pl.when(kb >= NKB - 2)
                    def _():
                        dy_copy_dyn(row_next, kb - (NKB - 2), slot).start()

                b = w_bf[:, pl.ds(kb * KB, KB)]
                part = lax.dot_general(
                    a, b, (((1,), (1,)), ((), ())),
                    preferred_element_type=jnp.float32,
                )

                @pl.when(kb == 0)
                def _():
                    acc[...] = part

                @pl.when(kb != 0)
                def _():
                    acc[...] += part

                return 0

            lax.fori_loop(0, NKB, kb_body, 0)

            if c == 0:
                comm_r[0, :, :] = acc[0:HT, :].astype(jnp.bfloat16)
                comm_l[0, :, :] = acc[HT:C, :].astype(jnp.bfloat16)
                if not _SKIP_RING:
                    for d in hop(0):
                        d.start()
            else:
                part_bf[(c - 1) * C:c * C, :] = acc[...].astype(jnp.bfloat16)

            if _SKIP_RING and c > 0:
                store_half(part_bf[(c - 1) * C:(c - 1) * C + HT, :],
                           ords[c] * C, 0)
                store_half(part_bf[(c - 1) * C + HT:c * C, :],
                           ords[c] * C + HT, 1)

            if not _SKIP_RING and c == 2:
                for d in hop(0):
                    d.wait()
                comm_r[1, :, :] += part_bf[0:HT, :]
                comm_l[1, :, :] += part_bf[C + HT:2 * C, :]
                for d in hop(1):
                    d.start()

        if _SKIP_RING:
            store_half(comm_r[0, :, :], ords[0] * C, 0)
            store_half(comm_l[0, :, :], ords[0] * C + HT, 1)
            pending_out[0].wait()
            pending_out[1].wait()
            return

        for d in hop(1):
            d.wait()
        comm_r[0, :, :] += part_bf[2 * C:2 * C + HT, :]
        comm_l[0, :, :] += part_bf[2 * C + HT:3 * C, :]

        h2 = hop(2)
        for d in h2:
            d.start()
        h2[0].wait_recv()
        comm_r[1, 0:SB, :] += part_bf[C:C + SB, :]
        h2[1].wait_recv()
        comm_l[1, 0:SB, :] += part_bf[HT:HT + SB, :]
        a3 = hop(3)
        a3[0].start()
        a3[1].start()
        h2[2].wait_recv()
        comm_r[1, SB:HT, :] += part_bf[C + SB:C + HT, :]
        h2[3].wait_recv()
        comm_l[1, SB:HT, :] += part_bf[HT + SB:C, :]
        a3[2].start()
        a3[3].start()
        store_half(comm_r[1, :, :], lax.rem(p + 1, N_DEV) * C, 0)
        store_half(comm_l[1, :, :], lax.rem(p + N_DEV - 1, N_DEV) * C + HT, 1)

        prev, prevprev = a3, h2
        for h in (4, 5):
            g = h - N_DEV
            nxt = hop(h)
            rs = h % 2
            for i in range(4):
                prev[i].wait_recv()
                prevprev[i].wait_send()
                nxt[i].start()
            idx_r = lax.rem(p + 2 * N_DEV - g, N_DEV)
            idx_l = lax.rem(p + g, N_DEV)
            store_half(comm_r[rs, :, :], idx_r * C, 0)
            store_half(comm_l[rs, :, :], idx_l * C + HT, 1)
            prev, prevprev = nxt, prev

        for d in prev:
            d.wait_recv()
        idx_r = lax.rem(p + 2 * N_DEV - 2, N_DEV)
        idx_l = lax.rem(p + 2, N_DEV)
        store_half(comm_r[0, :, :], idx_r * C, 0)
        store_half(comm_l[0, :, :], idx_l * C + HT, 1)

        for d in prevprev:
            d.wait_send()
        for d in prev:
            d.wait_send()

        pending_out[0].wait()
        pending_out[1].wait()

    return pl.pallas_call(
        body,
        out_shape=jax.ShapeDtypeStruct((M, N), jnp.float32),
        in_specs=[
            pl.BlockSpec(memory_space=pl.ANY),
            pl.BlockSpec(memory_space=pl.ANY),
        ],
        out_specs=pl.BlockSpec(memory_space=pl.ANY),
        scratch_shapes=[
            pltpu.VMEM((N, KSH), jnp.bfloat16),
            pltpu.VMEM((2, N, WB), jnp.float32),
            pltpu.VMEM((2, C, KB), jnp.float32),
            pltpu.VMEM((C, N), jnp.float32),
            pltpu.VMEM((3 * C, N), jnp.bfloat16),
            pltpu.VMEM((2, HT, N), jnp.bfloat16),
            pltpu.VMEM((2, HT, N), jnp.bfloat16),
            pltpu.VMEM((2, HT, N), jnp.float32),
            pltpu.SemaphoreType.DMA((2,)),
            pltpu.SemaphoreType.DMA((2,)),
            pltpu.SemaphoreType.DMA((2,)),
            pltpu.SemaphoreType.DMA((2, 2)),
            pltpu.SemaphoreType.DMA((2, 2)),
            pltpu.SemaphoreType.DMA((2, 2)),
            pltpu.SemaphoreType.DMA((2, 2)),
        ],
        compiler_params=pltpu.CompilerParams(
            collective_id=0,
            vmem_limit_bytes=63 * 1024 * 1024,
        ),
    )(dy, W)


def kernel(dy, W):
    return _fused_kernel(dy, W)
